# Initial kernel scaffold; baseline (speedup 1.0000x reference)
import jax
import jax.numpy as jnp
from jax.experimental import pallas as pl


def kernel(x, embedding, W1, b1, W2, b2, W3, b3, edge_index):
    raise NotImplementedError("write your pallas kernel here")


# speedup vs baseline: 535.5493x; 535.5493x over previous
import functools

import jax
import jax.numpy as jnp
from jax import lax
from jax.experimental import pallas as pl
from jax.experimental.pallas import tpu as pltpu
from jax.experimental.pallas import tpu_sc as plsc

NUM = 1024
D = 64
H_DIM = 64
E = 16384
MAIN = 16
LANES = 16

NC, NS = 2, 16
NW = NC * NS
ROWS = 16
PASSES = NUM // (NW * ROWS)
CHUNKS = E // LANES


def _build_counts(edge_index):
    mesh = plsc.VectorSubcoreMesh(core_axis_name="c", subcore_axis_name="s")

    @functools.partial(
        pl.kernel,
        mesh=mesh,
        out_type=jax.ShapeDtypeStruct((NUM * NUM,), jnp.float32),
        compiler_params=pltpu.CompilerParams(needs_layout_passes=False),
        scratch_types=[
            pltpu.VMEM((E,), jnp.int32),
            pltpu.VMEM((E,), jnp.int32),
            pltpu.VMEM((ROWS * NUM,), jnp.float32),
        ],
    )
    def sc_kernel(edge_hbm, out_hbm, src_v, dst_v, blk_v):
        wid = lax.axis_index("s") * NC + lax.axis_index("c")
        pltpu.sync_copy(edge_hbm.at[0], src_v)
        pltpu.sync_copy(edge_hbm.at[1], dst_v)
        ones = jnp.full((LANES,), 1.0, jnp.float32)
        zeros = jnp.zeros((LANES,), jnp.float32)
        for p in range(PASSES):
            base = (p * NW + wid) * ROWS

            def zero_body(i, carry):
                blk_v[pl.ds(i * LANES, LANES)] = zeros
                return carry

            lax.fori_loop(0, ROWS * NUM // LANES, zero_body, 0)

            def edge_body(i, carry):
                d16 = dst_v[pl.ds(i * LANES, LANES)]
                s16 = src_v[pl.ds(i * LANES, LANES)]
                m = (d16 >= base) & (d16 < base + ROWS)
                idx = (d16 - base) * NUM + s16
                plsc.addupdate_scatter(blk_v, [idx], ones, mask=m)
                return carry

            lax.fori_loop(0, CHUNKS, edge_body, 0)
            pltpu.sync_copy(blk_v, out_hbm.at[pl.ds(base * NUM, ROWS * NUM)])

    return sc_kernel(edge_index)


def _forward_body(xt_ref, emb_ref, w1_ref, b1_ref, w2_ref, b2_ref,
                  w3_ref, b3_ref, c_ref, out_ref):
    G = xt_ref.shape[1]
    W = G * H_DIM
    NB = 8
    S = NB * H_DIM
    Cm = c_ref[...]
    deg = jnp.sum(Cm, axis=1)
    dinv = jnp.where(deg > 0, lax.rsqrt(jnp.maximum(deg, 1e-12)), 0.0)
    dcol = dinv[:, None]
    xtb = xt_ref[...]

    def mm(a, b):
        return jnp.dot(a, b, preferred_element_type=jnp.float32)

    rep = (lax.broadcasted_iota(jnp.int32, (G, W), 1) // H_DIM
           == lax.broadcasted_iota(jnp.int32, (G, W), 0)).astype(jnp.float32)
    xe = mm(xtb, rep)
    ew1 = mm(emb_ref[...], jnp.tile(w1_ref[...], (1, G)))
    s1 = xe * ew1 * dcol
    h1 = mm(Cm, s1) * dcol + jnp.tile(b1_ref[...], G)[None, :]
    h1 = jnp.maximum(h1, 0.0)
    bmask = (lax.broadcasted_iota(jnp.int32, (S, S), 0) // H_DIM
             == lax.broadcasted_iota(jnp.int32, (S, S), 1) // H_DIM)
    bd2 = jnp.where(bmask, jnp.tile(w2_ref[...], (NB, NB)), 0.0)
    t2 = [mm(h1[:, i * S:(i + 1) * S], bd2) for i in range(W // S)]
    s2 = jnp.concatenate(t2, axis=1) * dcol
    h2 = mm(Cm, s2) * dcol + jnp.tile(b2_ref[...], G)[None, :]
    h2 = jnp.maximum(h2, 0.0)
    k3 = jnp.where(
        lax.broadcasted_iota(jnp.int32, (W, G), 0) // H_DIM
        == lax.broadcasted_iota(jnp.int32, (W, G), 1),
        jnp.tile(w3_ref[...], (G, 1)), 0.0)
    s3 = mm(h2, k3) * dcol
    logits = mm(Cm, s3) * dcol + b3_ref[0]
    sm = logits[:MAIN, :]
    mx = jnp.max(sm, axis=0, keepdims=True)
    ex = jnp.exp(sm - mx)
    smx = ex / jnp.sum(ex, axis=0, keepdims=True)
    sig = 1.0 / (1.0 + jnp.exp(-logits[MAIN:, :]))
    out_ref[...] = jnp.concatenate([smx, sig], axis=0)


def _forward(xt, embedding, W1, b1, W2, b2, W3, b3, C):
    B = xt.shape[1]
    return pl.pallas_call(
        _forward_body,
        out_shape=jax.ShapeDtypeStruct((NUM, B), jnp.float32),
    )(xt, embedding, W1, b1, W2, b2, W3, b3, C)


def kernel(x, embedding, W1, b1, W2, b2, W3, b3, edge_index):
    C = _build_counts(edge_index).reshape(NUM, NUM)
    phi_nm = _forward(x.T, embedding, W1, b1, W2, b2, W3, b3, C)
    return phi_nm.T


# speedup vs baseline: 698.2010x; 1.3037x over previous
import functools

import jax
import jax.numpy as jnp
from jax import lax
from jax.experimental import pallas as pl
from jax.experimental.pallas import tpu as pltpu
from jax.experimental.pallas import tpu_sc as plsc

NUM = 1024
D = 64
H_DIM = 64
E = 16384
MAIN = 16
LANES = 16

NC, NS = 2, 16
NW = NC * NS
ROWS = 32
CHUNKS = E // LANES


def _build_counts(edge_index):
    mesh = plsc.VectorSubcoreMesh(core_axis_name="c", subcore_axis_name="s")

    @functools.partial(
        pl.kernel,
        mesh=mesh,
        out_type=jax.ShapeDtypeStruct((NUM, NUM), jnp.float32),
        compiler_params=pltpu.CompilerParams(needs_layout_passes=False),
        scratch_types=[
            pltpu.VMEM((E,), jnp.int32),
            pltpu.VMEM((E,), jnp.int32),
            pltpu.VMEM((ROWS, NUM), jnp.float32),
        ],
    )
    def sc_kernel(edge_hbm, out_hbm, src_v, dst_v, blk_v):
        wid = lax.axis_index("s") * NC + lax.axis_index("c")
        pltpu.sync_copy(edge_hbm.at[0], src_v)
        pltpu.sync_copy(edge_hbm.at[1], dst_v)
        ones = jnp.full((LANES,), 1.0, jnp.float32)
        zeros = jnp.zeros((LANES,), jnp.float32)
        base = wid * ROWS
        UN = 8

        def zero_body(i, carry):
            for u in range(NUM // (LANES * UN)):
                for v in range(UN):
                    blk_v[i, pl.ds((u * UN + v) * LANES, LANES)] = zeros
            return carry

        lax.fori_loop(0, ROWS, zero_body, 0)

        UNE = 16

        def edge_body(i, carry):
            for u in range(UNE):
                off = (i * UNE + u) * LANES
                d16 = dst_v[pl.ds(off, LANES)]
                s16 = src_v[pl.ds(off, LANES)]
                m = (d16 >= base) & (d16 < base + ROWS)
                plsc.addupdate_scatter(blk_v, [d16 - base, s16], ones, mask=m)
            return carry

        lax.fori_loop(0, CHUNKS // UNE, edge_body, 0)

        pltpu.sync_copy(blk_v, out_hbm.at[pl.ds(base, ROWS)])

    return sc_kernel(edge_index)


def _forward_body(x_ref, emb_ref, w1_ref, b1_ref, w2_ref, b2_ref,
                  w3_ref, b3_ref, c_ref, out_ref):
    G = x_ref.shape[0]
    W = G * H_DIM
    NB = 8
    S = NB * H_DIM
    Cm = c_ref[...]
    deg = jnp.sum(Cm, axis=1)
    dinv = jnp.where(deg > 0, lax.rsqrt(jnp.maximum(deg, 1e-12)), 0.0)
    dcol = dinv[:, None]
    xtb = x_ref[...].T

    def mm(a, b):
        return jnp.dot(a, b, preferred_element_type=jnp.float32)

    rep = (lax.broadcasted_iota(jnp.int32, (G, W), 1) // H_DIM
           == lax.broadcasted_iota(jnp.int32, (G, W), 0)).astype(jnp.float32)
    xe = mm(xtb, rep)
    ew1 = mm(emb_ref[...], jnp.tile(w1_ref[...], (1, G)))
    s1 = xe * ew1 * dcol
    h1 = mm(Cm, s1) * dcol + jnp.tile(b1_ref[...], G)[None, :]
    h1 = jnp.maximum(h1, 0.0)
    bmask = (lax.broadcasted_iota(jnp.int32, (S, S), 0) // H_DIM
             == lax.broadcasted_iota(jnp.int32, (S, S), 1) // H_DIM)
    bd2 = jnp.where(bmask, jnp.tile(w2_ref[...], (NB, NB)), 0.0)
    t2 = [mm(h1[:, i * S:(i + 1) * S], bd2) for i in range(W // S)]
    s2 = jnp.concatenate(t2, axis=1) * dcol
    h2 = mm(Cm, s2) * dcol + jnp.tile(b2_ref[...], G)[None, :]
    h2 = jnp.maximum(h2, 0.0)
    k3 = jnp.where(
        lax.broadcasted_iota(jnp.int32, (W, G), 0) // H_DIM
        == lax.broadcasted_iota(jnp.int32, (W, G), 1),
        jnp.tile(w3_ref[...], (G, 1)), 0.0)
    s3 = mm(h2, k3) * dcol
    logits = mm(Cm, s3) * dcol + b3_ref[0]
    sm = logits[:MAIN, :]
    mx = jnp.max(sm, axis=0, keepdims=True)
    ex = jnp.exp(sm - mx)
    smx = ex / jnp.sum(ex, axis=0, keepdims=True)
    sig = 1.0 / (1.0 + jnp.exp(-logits[MAIN:, :]))
    out_ref[...] = jnp.concatenate([smx, sig], axis=0).T


def _forward(x, embedding, W1, b1, W2, b2, W3, b3, C):
    B = x.shape[0]
    return pl.pallas_call(
        _forward_body,
        out_shape=jax.ShapeDtypeStruct((B, NUM), jnp.float32),
    )(x, embedding, W1, b1, W2, b2, W3, b3, C)


def kernel(x, embedding, W1, b1, W2, b2, W3, b3, edge_index):
    C = _build_counts(edge_index)
    return _forward(x, embedding, W1, b1, W2, b2, W3, b3, C)
